# baseline (speedup 1.0000x reference)
import functools

import jax
import jax.numpy as jnp
from jax import lax
from jax.experimental import pallas as pl
from jax.experimental.pallas import tpu as pltpu
from jax.experimental.pallas import tpu_sc as plsc

_NC = 2
_NS = 16
_NW = _NC * _NS
_BPC = 2


@functools.lru_cache(maxsize=None)
def _build(B, H, D, NBUF):
    b_per_w = B // _NW
    CH = _BPC * H
    n_chunks = b_per_w // _BPC
    n_groups = n_chunks // NBUF
    HALF = NBUF // 2
    assert n_chunks % NBUF == 0 and n_groups >= 2 and CH <= 128
    mesh = plsc.VectorSubcoreMesh(core_axis_name="c", subcore_axis_name="s")

    @functools.partial(
        pl.kernel,
        mesh=mesh,
        out_type=jax.ShapeDtypeStruct((B, H, D), jnp.float32),
        compiler_params=pltpu.CompilerParams(use_tc_tiling_on_sc=False),
        scratch_types=[
            pltpu.VMEM((n_chunks, CH), jnp.int32),
            pltpu.VMEM((NBUF, CH, D), jnp.float32),
            pltpu.SemaphoreType.DMA((NBUF,)),
            pltpu.SemaphoreType.DMA((NBUF,)),
        ],
    )
    def emb(idx_hbm, table_hbm, out_hbm, idx_v, rows_v, gsem, wsem):
        wid = lax.axis_index("s") * _NC + lax.axis_index("c")
        base_b = wid * b_per_w
        pltpu.sync_copy(idx_hbm.at[wid], idx_v)

        def gather(chunk, buf):
            pltpu.async_copy(table_hbm.at[idx_v.at[chunk]], rows_v.at[buf],
                             gsem.at[buf])

        def gather_wait(chunk, buf):
            pltpu.make_async_copy(table_hbm.at[idx_v.at[chunk]],
                                  rows_v.at[buf], gsem.at[buf]).wait()

        def wb(chunk, buf):
            b0 = base_b + _BPC * chunk
            for k in range(_BPC):
                pltpu.async_copy(rows_v.at[buf, pl.ds(k * H, H)],
                                 out_hbm.at[b0 + k], wsem.at[buf])

        def wb_wait(buf):
            for k in range(_BPC):
                pltpu.make_async_copy(rows_v.at[buf, pl.ds(k * H, H)],
                                      out_hbm.at[base_b], wsem.at[buf]).wait()

        for c in range(HALF):
            gather(c, c)

        def group(g, carry):
            for b in range(NBUF):
                j = g * NBUF + b
                b2 = (b + HALF) % NBUF
                if b < HALF:
                    @pl.when(g > 0)
                    def _():
                        wb_wait(b2)
                    gather(j + HALF, b2)
                else:
                    wb_wait(b2)

                    @pl.when(g < n_groups - 1)
                    def _():
                        gather(j + HALF, b2)
                gather_wait(j, b)
                wb(j, b)
            return carry

        lax.fori_loop(0, n_groups, group, 0)

        for c in range(n_chunks - HALF, n_chunks):
            wb_wait(c % NBUF)

    return emb


def kernel(X, table):
    B, H = X.shape
    V, D = table.shape
    idx = X.reshape(_NW, (B // _NW) // _BPC, _BPC * H)
    return _build(B, H, D, 8)(idx, table)
